# baseline (speedup 1.0000x reference)
import jax
import jax.numpy as jnp
from jax import lax
from jax.experimental import pallas as pl
from jax.experimental.pallas import tpu as pltpu
from jax.experimental.pallas import tpu_sc as plsc

DAT_NUM = 1000000
DEL_NUM = 32
BATCH = 16384

_NC = 2
_NS = 16
_NW = _NC * _NS
_BPW = BATCH // _NW
_CI = 48
_NCHUNK = _BPW // _CI + 1
_NG = (_CI + 15) // 16
_WPC = _CI * DEL_NUM
_NJ = _WPC // 128
_VPG = 32


def _body(idx_hbm, tableT_hbm, out_hbm, idx_v, addr_a, addr_b, dst_a,
          dst_b, stage_v, sem):
    wid = lax.axis_index("s") * _NC + lax.axis_index("c")
    base = wid * _BPW
    words_hbm = tableT_hbm.at[0]

    pltpu.sync_copy(idx_hbm.at[pl.ds(base, _BPW)], idx_v)
    c16 = lax.iota(jnp.int32, 16)

    def bc16(s):
        return jnp.broadcast_to(s, (16,)).astype(jnp.int32)

    def build(k, slot):
        addr_v = addr_a if slot == 0 else addr_b
        dst_v = dst_a if slot == 0 else dst_b

        def grp(g, carry):
            i0 = jnp.minimum(k * _CI + g * 16, _BPW - 16)
            r16 = idx_v[pl.ds(pl.multiple_of(i0, 16), 16)]
            grp_row = g * 16 * DEL_NUM
            for c in range(DEL_NUM):
                w16 = c * DAT_NUM + r16
                row16 = grp_row + c16 * DEL_NUM + c
                plsc.store_scatter(
                    addr_v, [row16 // 128, lax.rem(row16, 128)], w16
                )
            return carry

        lax.fori_loop(0, _NG, grp, 0)
        for j in range(_NJ):
            pltpu.async_copy(
                words_hbm.at[addr_v.at[j]], dst_v.at[j], sem
            )

    def drain(slot):
        addr_v = addr_a if slot == 0 else addr_b
        dst_v = dst_a if slot == 0 else dst_b
        for j in range(_NJ):
            pltpu.make_async_copy(
                words_hbm.at[addr_v.at[j]], dst_v.at[j], sem
            ).wait()

    def extract(k, slot):
        dst_v = dst_a if slot == 0 else dst_b

        def grp(g, carry):
            i0 = jnp.minimum(k * _CI + g * 16, _BPW - 16)
            for h in range(_VPG):
                v = dst_v[g * 4 + h // 8, pl.ds((h % 8) * 16, 16)]
                s = 1.0 / (1.0 + jnp.exp(-v))
                plsc.store_scatter(
                    stage_v,
                    [c16 + (h % 2) * 16, bc16(i0 + h // 2)],
                    s,
                )
            return carry

        lax.fori_loop(0, _NG, grp, 0)

    build(0, 0)
    for k in range(_NCHUNK):
        if k + 1 < _NCHUNK:
            build(k + 1, (k + 1) % 2)
        drain(k % 2)
        extract(k, k % 2)

    copies = []
    for c in range(DEL_NUM):
        copies.append(pltpu.async_copy(
            stage_v.at[c],
            out_hbm.at[pl.ds(c * BATCH + base, _BPW)],
            sem,
        ))
    for cp in copies:
        cp.wait()


@jax.jit
def _sc_lookup_sigmoid(idx, table):
    mesh = plsc.VectorSubcoreMesh(core_axis_name="c", subcore_axis_name="s")
    k = pl.kernel(
        _body,
        out_type=jax.ShapeDtypeStruct((DEL_NUM * BATCH,), jnp.float32),
        mesh=mesh,
        scratch_types=[
            pltpu.VMEM((_BPW,), jnp.int32),
            pltpu.VMEM((_NJ, 128), jnp.int32),
            pltpu.VMEM((_NJ, 128), jnp.int32),
            pltpu.VMEM((_NJ, 128), jnp.float32),
            pltpu.VMEM((_NJ, 128), jnp.float32),
            pltpu.VMEM((DEL_NUM, _BPW), jnp.float32),
            pltpu.SemaphoreType.DMA,
        ],
        compiler_params=pltpu.CompilerParams(
            needs_layout_passes=False, use_tc_tiling_on_sc=False
        ),
    )
    return k(idx, table.T)


def kernel(idx, table):
    flat = _sc_lookup_sigmoid(idx.astype(jnp.int32), table)
    return flat.reshape(DEL_NUM, BATCH).T[:, :, None]


# speedup vs baseline: 4.9291x; 4.9291x over previous
import jax
import jax.numpy as jnp
from jax import lax
from jax.experimental import pallas as pl
from jax.experimental.pallas import tpu as pltpu
from jax.experimental.pallas import tpu_sc as plsc

DAT_NUM = 1000000
DEL_NUM = 32
BATCH = 16384

_NC = 2
_NS = 16
_NW = _NC * _NS
_BPW = BATCH // _NW
_CHUNK = 128
_NCHUNK = _BPW // _CHUNK


def _sc_body(idx_hbm, table_hbm, out_hbm, idx_v, rows_v, sem):
    wid = lax.axis_index("s") * _NC + lax.axis_index("c")
    base = wid * _BPW

    pltpu.sync_copy(idx_hbm.at[pl.ds(wid * _NCHUNK, _NCHUNK)], idx_v)

    copies = []
    for j in range(_NCHUNK):
        copies.append(
            pltpu.async_copy(
                table_hbm.at[idx_v.at[j]],
                rows_v.at[pl.ds(j * _CHUNK, _CHUNK)],
                sem,
            )
        )
    for c in copies:
        c.wait()

    def body(i, carry):
        for h in (0, 16):
            v = rows_v[i, pl.ds(h, 16)]
            rows_v[i, pl.ds(h, 16)] = 1.0 / (1.0 + jnp.exp(-v))
        return carry

    lax.fori_loop(0, _BPW, body, 0, unroll=4)

    pltpu.sync_copy(rows_v, out_hbm.at[pl.ds(base, _BPW)])


@jax.jit
def _sc_lookup_sigmoid(idx, table):
    mesh = plsc.VectorSubcoreMesh(core_axis_name="c", subcore_axis_name="s")
    k = pl.kernel(
        _sc_body,
        out_type=jax.ShapeDtypeStruct((BATCH, DEL_NUM), jnp.float32),
        mesh=mesh,
        scratch_types=[
            pltpu.VMEM((_NCHUNK, _CHUNK), jnp.int32),
            pltpu.VMEM((_BPW, DEL_NUM), jnp.float32),
            pltpu.SemaphoreType.DMA,
        ],
        compiler_params=pltpu.CompilerParams(use_tc_tiling_on_sc=False),
    )
    return k(idx.reshape(_NW * _NCHUNK, _CHUNK), table)


def kernel(idx, table):
    out = _sc_lookup_sigmoid(idx.astype(jnp.int32), table)
    return out[:, :, None]
